# baseline (speedup 1.0000x reference)
import jax
import jax.numpy as jnp
from jax.experimental import pallas as pl
from jax.experimental.pallas import tpu as pltpu


def _mm_nn(a, b):
    return jax.lax.dot_general(a, b, (((1,), (0,)), ((), ())),
                               preferred_element_type=jnp.float32)


def _mm_nt(a, b):
    return jax.lax.dot_general(a, b, (((1,), (1,)), ((), ())),
                               preferred_element_type=jnp.float32)


def _onehot_bf16(idx_col, n):
    cols = jax.lax.broadcasted_iota(jnp.int32, (idx_col.shape[0], n), 1)
    return jnp.where(cols == idx_col, 1.0, 0.0).astype(jnp.bfloat16)


def _group_matrix(f):
    r = jax.lax.broadcasted_iota(jnp.int32, (7 * f, f), 0)
    c = jax.lax.broadcasted_iota(jnp.int32, (7 * f, f), 1)
    u = (r - c * 7).astype(jnp.uint32)
    return jnp.where(u < 7, 1.0, 0.0).astype(jnp.float32)


def _gather7(h, es, m):
    n, f = h.shape
    hi = h.astype(jnp.bfloat16)
    lo = (h - hi.astype(jnp.float32)).astype(jnp.bfloat16)
    hcat = jnp.concatenate([hi, lo], axis=1)
    blocks = []
    for d in range(6):
        g = _mm_nn(es[d][0:m, :], hcat)
        blocks.append(g[:, 0:f] + g[:, f:2 * f])
    blocks.append(h[0:m, :])
    return jnp.concatenate(blocks, axis=1)


def _bn_lrelu(h, g, be):
    mu = jnp.mean(h, axis=0, keepdims=True)
    c = h - mu
    var = jnp.mean(c * c, axis=0, keepdims=True)
    y = c * jax.lax.rsqrt(var + 1e-5) * g + be
    return jnp.where(y >= 0.0, y, 0.2 * y)


def _row(ref):
    return ref[...].reshape(1, -1)


def _body(x_ref, w1_ref, b1_ref, g1_ref, be1_ref, w2_ref, b2_ref, g2_ref,
          be2_ref, w3_hbm, b3_ref, g3_ref, be3_ref, wfc_ref, bfc_ref,
          no_ref, out_ref, w3_v, sem3):
    cp3 = pltpu.make_async_copy(w3_hbm, w3_v, sem3)
    cp3.start()

    x = x_ref[...]
    no = no_ref[...]
    no1 = no[0:642, :]
    no2 = no[642:804, :]
    no3 = no[804:846, :]

    es1 = [_onehot_bf16(no1[:, d:d + 1], 642) for d in range(6)]
    es2 = [_onehot_bf16(no2[:, d:d + 1], 162) for d in range(6)]
    es3 = [_onehot_bf16(no3[:, d:d + 1], 42) for d in range(6)]

    h = _mm_nt(_gather7(x, es1, 642), w1_ref[...]) + _row(b1_ref)
    h = _bn_lrelu(h, _row(g1_ref), _row(be1_ref))
    h = _mm_nn(_gather7(h, es1, 162), _group_matrix(128)) * (1.0 / 7.0)

    h = _mm_nt(_gather7(h, es2, 162), w2_ref[...]) + _row(b2_ref)
    h = _bn_lrelu(h, _row(g2_ref), _row(be2_ref))
    h = _mm_nn(_gather7(h, es2, 42), _group_matrix(256)) * (1.0 / 7.0)

    cp3.wait()
    h = _mm_nt(_gather7(h, es3, 42), w3_v[...]) + _row(b3_ref)
    h = _bn_lrelu(h, _row(g3_ref), _row(be3_ref))
    h = _mm_nn(_gather7(h, es3, 12), _group_matrix(512)) * (1.0 / 7.0)

    t = h * wfc_ref[...]
    s = jnp.sum(t) * (1.0 / 12.0) + bfc_ref[0]
    sv = jnp.full((1, 1), s, jnp.float32)
    out_ref[...] = 1.0 / (1.0 + jnp.exp(-sv))


def kernel(x, W1, b1, g1, be1, W2, b2, g2, be2, W3, b3, g3, be3, Wfc, bfc,
           no642, no162, no42):
    noall = jnp.concatenate([no642.astype(jnp.int32), no162.astype(jnp.int32),
                             no42.astype(jnp.int32)]).reshape(846, 7)
    vspec = pl.BlockSpec(memory_space=pltpu.VMEM)
    aspec = pl.BlockSpec(memory_space=pltpu.HBM)
    sspec = pl.BlockSpec(memory_space=pltpu.SMEM)
    out = pl.pallas_call(
        _body,
        out_shape=jax.ShapeDtypeStruct((1, 1), jnp.float32),
        in_specs=[vspec, vspec, vspec, vspec, vspec,
                  vspec, vspec, vspec, vspec,
                  aspec, vspec, vspec, vspec,
                  vspec, sspec, vspec],
        out_specs=vspec,
        scratch_shapes=[
            pltpu.VMEM((512, 1792), jnp.float32),
            pltpu.SemaphoreType.DMA,
        ],
    )(x, W1, b1, g1, be1, W2, b2, g2, be2, W3, b3, g3, be3,
      Wfc, bfc.reshape(1), noall)
    return out.reshape(1)


# speedup vs baseline: 1.0203x; 1.0203x over previous
import jax
import jax.numpy as jnp
from jax.experimental import pallas as pl
from jax.experimental.pallas import tpu as pltpu


def _mm_nn(a, b):
    return jax.lax.dot_general(a, b, (((1,), (0,)), ((), ())),
                               preferred_element_type=jnp.float32)


def _mm_nt(a, b):
    return jax.lax.dot_general(a, b, (((1,), (1,)), ((), ())),
                               preferred_element_type=jnp.float32)


def _onehot_bf16(idx_col, n):
    cols = jax.lax.broadcasted_iota(jnp.int32, (idx_col.shape[0], n), 1)
    return jnp.where(cols == idx_col, 1.0, 0.0).astype(jnp.bfloat16)


def _group_matrix(f):
    r = jax.lax.broadcasted_iota(jnp.int32, (7 * f, f), 0)
    c = jax.lax.broadcasted_iota(jnp.int32, (7 * f, f), 1)
    u = (r - c * 7).astype(jnp.uint32)
    return jnp.where(u < 7, 1.0, 0.0).astype(jnp.float32)


def _gather7(h, es, m):
    n, f = h.shape
    hi = h.astype(jnp.bfloat16)
    lo = (h - hi.astype(jnp.float32)).astype(jnp.bfloat16)
    hcat = jnp.concatenate([hi, lo], axis=1)
    blocks = []
    for d in range(6):
        g = _mm_nn(es[d][0:m, :], hcat)
        blocks.append(g[:, 0:f] + g[:, f:2 * f])
    blocks.append(h[0:m, :])
    return jnp.concatenate(blocks, axis=1)


def _bn_lrelu(h, g, be):
    mu = jnp.mean(h, axis=0, keepdims=True)
    c = h - mu
    var = jnp.mean(c * c, axis=0, keepdims=True)
    y = c * jax.lax.rsqrt(var + 1e-5) * g + be
    return jnp.where(y >= 0.0, y, 0.2 * y)


def _row(ref):
    return ref[...].reshape(1, -1)


def _body(x_ref, w1_ref, b1_ref, g1_ref, be1_ref, w2_hbm, b2_ref, g2_ref,
          be2_ref, w3_hbm, b3_ref, g3_ref, be3_ref, wfc_ref, bfc_ref,
          no_ref, out_ref, w2_v, w3_v, sem2, sem3):
    cp2 = pltpu.make_async_copy(w2_hbm, w2_v, sem2)
    cp3 = pltpu.make_async_copy(w3_hbm, w3_v, sem3)
    cp2.start()
    cp3.start()

    x = x_ref[...]
    no = no_ref[...]
    no1 = no[0:642, :]
    no2 = no[642:804, :]
    no3 = no[804:846, :]

    es1 = [_onehot_bf16(no1[:, d:d + 1], 642) for d in range(6)]
    es2 = [_onehot_bf16(no2[:, d:d + 1], 162) for d in range(6)]
    es3 = [_onehot_bf16(no3[:, d:d + 1], 42) for d in range(6)]

    h = _mm_nt(_gather7(x, es1, 642), w1_ref[...]) + _row(b1_ref)
    h = _bn_lrelu(h, _row(g1_ref), _row(be1_ref))
    h = _mm_nn(_gather7(h, es1, 162), _group_matrix(128)) * (1.0 / 7.0)

    cp2.wait()
    h = _mm_nt(_gather7(h, es2, 162), w2_v[...]) + _row(b2_ref)
    h = _bn_lrelu(h, _row(g2_ref), _row(be2_ref))
    h = _mm_nn(_gather7(h, es2, 42), _group_matrix(256)) * (1.0 / 7.0)

    cp3.wait()
    h = _mm_nt(_gather7(h, es3, 42), w3_v[...]) + _row(b3_ref)
    h = _bn_lrelu(h, _row(g3_ref), _row(be3_ref))
    h = _mm_nn(_gather7(h, es3, 12), _group_matrix(512)) * (1.0 / 7.0)

    t = h * wfc_ref[...]
    s = jnp.sum(t) * (1.0 / 12.0) + bfc_ref[0]
    sv = jnp.full((1, 1), s, jnp.float32)
    out_ref[...] = 1.0 / (1.0 + jnp.exp(-sv))


def kernel(x, W1, b1, g1, be1, W2, b2, g2, be2, W3, b3, g3, be3, Wfc, bfc,
           no642, no162, no42):
    noall = jnp.concatenate([no642.astype(jnp.int32), no162.astype(jnp.int32),
                             no42.astype(jnp.int32)]).reshape(846, 7)
    vspec = pl.BlockSpec(memory_space=pltpu.VMEM)
    aspec = pl.BlockSpec(memory_space=pltpu.HBM)
    sspec = pl.BlockSpec(memory_space=pltpu.SMEM)
    out = pl.pallas_call(
        _body,
        out_shape=jax.ShapeDtypeStruct((1, 1), jnp.float32),
        in_specs=[vspec, vspec, vspec, vspec, vspec,
                  aspec, vspec, vspec, vspec,
                  aspec, vspec, vspec, vspec,
                  vspec, sspec, vspec],
        out_specs=vspec,
        scratch_shapes=[
            pltpu.VMEM((256, 896), jnp.float32),
            pltpu.VMEM((512, 1792), jnp.float32),
            pltpu.SemaphoreType.DMA,
            pltpu.SemaphoreType.DMA,
        ],
    )(x, W1, b1, g1, be1, W2, b2, g2, be2, W3, b3, g3, be3,
      Wfc, bfc.reshape(1), noall)
    return out.reshape(1)
